# baseline (speedup 1.0000x reference)
import jax
import jax.numpy as jnp
from jax.experimental import pallas as pl
from jax.experimental.pallas import tpu as pltpu

E = 16
K = 2
D = 1024
O = 1024
A = 8192
M = 1024
NT = A // M
NI = NT + E - 1
BALANCE_W = 0.01


def _gmm_body(tiles, experts, rs_arr, re_arr, x_ref, w_ref, o_ref):
    del experts
    i = pl.program_id(0)
    t_prev = tiles[jnp.maximum(i - 1, 0)]
    first = jnp.logical_or(i == 0, tiles[i] != t_prev)
    rs = rs_arr[i]
    re = re_arr[i]

    @pl.when(first)
    def _():
        o_ref[...] = jnp.zeros_like(o_ref)

    @pl.when(rs < re)
    def _():
        rows = jax.lax.broadcasted_iota(jnp.int32, (M, 1), 0)
        mask = (rows >= rs) & (rows < re)
        xm = jnp.where(mask, x_ref[...], jnp.zeros_like(x_ref))
        o_ref[...] += jnp.dot(
            xm, w_ref[0], preferred_element_type=jnp.float32
        ).astype(o_ref.dtype)


def _grouped_matmul(xs, wt, tiles, experts, rs_arr, re_arr):
    grid_spec = pltpu.PrefetchScalarGridSpec(
        num_scalar_prefetch=4,
        grid=(NI,),
        in_specs=[
            pl.BlockSpec((M, D), lambda i, t, e, rs, re: (t[i], 0)),
            pl.BlockSpec((1, D, O), lambda i, t, e, rs, re: (e[i], 0, 0)),
        ],
        out_specs=pl.BlockSpec((M, O), lambda i, t, e, rs, re: (t[i], 0)),
    )
    return pl.pallas_call(
        _gmm_body,
        grid_spec=grid_spec,
        out_shape=jax.ShapeDtypeStruct((A, O), jnp.bfloat16),
    )(tiles, experts, rs_arr, re_arr, xs, wt)


def _cv_sq(v):
    return jnp.var(v, ddof=1) / (jnp.mean(v) ** 2 + 1e-10)


def kernel(x, gate_W, expert_W, expert_b):
    B, S, _ = x.shape
    xf = x.reshape(-1, D)

    logits = xf @ gate_W.T
    top_logits, top_idx = jax.lax.top_k(logits, K)
    top_scores = jax.nn.softmax(top_logits.astype(jnp.float32), axis=1)

    e_flat = top_idx.reshape(-1)
    s_flat = top_scores.reshape(-1)

    importance = jnp.zeros((E,), jnp.float32).at[e_flat].add(s_flat)
    load = jnp.zeros((E,), jnp.float32).at[e_flat].add(
        (s_flat > 0).astype(jnp.float32))
    gate_loss = (_cv_sq(importance) + _cv_sq(load)) * BALANCE_W

    sortidx = jnp.argsort(e_flat)
    pos = jnp.zeros((A,), jnp.int32).at[sortidx].set(
        jnp.arange(A, dtype=jnp.int32))
    counts = jnp.zeros((E,), jnp.int32).at[e_flat].add(1)
    offs = jnp.concatenate(
        [jnp.zeros((1,), jnp.int32), jnp.cumsum(counts).astype(jnp.int32)])

    ts = jnp.arange(NT, dtype=jnp.int32) * M
    st = jnp.maximum(ts[:, None], offs[None, :E])
    en = jnp.minimum(ts[:, None] + M, offs[None, 1:])
    valid = (en > st).reshape(-1)
    cum = jnp.cumsum(valid.astype(jnp.int32))
    total = cum[-1]
    pair = jnp.searchsorted(cum, jnp.arange(NI, dtype=jnp.int32) + 1)
    pair = jnp.minimum(pair, NT * E - 1).astype(jnp.int32)
    ivalid = jnp.arange(NI) < total
    t_i = (pair // E).astype(jnp.int32)
    e_i = (pair % E).astype(jnp.int32)
    rs = jnp.where(ivalid, st.reshape(-1)[pair] - t_i * M, 0).astype(jnp.int32)
    re_ = jnp.where(ivalid, en.reshape(-1)[pair] - t_i * M, 0).astype(jnp.int32)

    xs = (xf[sortidx // K] * s_flat[sortidx][:, None]).astype(jnp.bfloat16)
    wt = expert_W.transpose(0, 2, 1).astype(jnp.bfloat16)

    rows = _grouped_matmul(xs, wt, t_i, e_i, rs, re_)

    p0 = pos[0::K]
    p1 = pos[1::K]
    bias = (top_scores[:, 0:1] * expert_b[top_idx[:, 0]]
            + top_scores[:, 1:2] * expert_b[top_idx[:, 1]])
    y = rows[p0].astype(jnp.float32) + rows[p1].astype(jnp.float32) + bias
    return y.reshape(B, S, O), gate_loss
